# baseline (speedup 1.0000x reference)
import functools

import jax
import jax.numpy as jnp
from jax import lax
from jax.experimental import pallas as pl
from jax.experimental.pallas import tpu as pltpu
from jax.experimental.pallas import tpu_sc as plsc

N = 10000
F_IN = 128
HPAD = 16
BLK = 400
NBLK = N // BLK

NWORK = 32
SC_BASE = 7952
SC_ROWS = N - SC_BASE
TC_ROWS = 8400
TC_NBLK = TC_ROWS // BLK
RPW = SC_ROWS // NWORK
RB = 8
NEG = -3.0e38


def _pass1_body(x_ref, adj_ref, w1_ref, b1_ref, w2_ref,
                s2_ref, s2t_ref, s1_ref):
    i = pl.program_id(0)

    @pl.when(i == 0)
    def _():
        s1_ref[...] = jnp.dot(x_ref[...], w1_ref[...],
                              preferred_element_type=jnp.float32)

    t = jnp.dot(adj_ref[...], s1_ref[...],
                preferred_element_type=jnp.float32)
    h = jnp.maximum(t + b1_ref[...], 0.0)
    s2 = jnp.dot(h, w2_ref[...], preferred_element_type=jnp.float32)
    s2_ref[...] = s2
    s2t_ref[...] = s2.T[:2].reshape(1, 2, BLK)


def _pass2_tc_body(adj_ref, s2_ref, max_ref, acc_ref):
    i = pl.program_id(0)
    u = jnp.dot(adj_ref[...], s2_ref[...],
                preferred_element_type=jnp.float32)
    m = jnp.max(u, axis=0, keepdims=True)
    prev = jnp.where(i == 0, jnp.full((1, HPAD), NEG, jnp.float32),
                     acc_ref[0:1, :])
    acc_ref[0:1, :] = jnp.maximum(prev, m)

    @pl.when(i == TC_NBLK - 1)
    def _():
        max_ref[...] = jnp.broadcast_to(acc_ref[0:1, :], (8, HPAD))


def _lane_sum(v):
    lane = lax.iota(jnp.int32, 16)
    dnums = lax.GatherDimensionNumbers(
        offset_dims=(), collapsed_slice_dims=(0,), start_index_map=(0,))
    for k in (8, 4, 2, 1):
        v = v + lax.gather(v, (lane ^ k)[:, None], dnums, (1,),
                           mode=lax.GatherScatterMode.PROMISE_IN_BOUNDS)
    return v


def _pass2_sc_body(adj_hbm, s2t_hbm, out_hbm, c01_ref, abuf_ref, out_ref):
    wid = lax.axis_index("s") * 2 + lax.axis_index("c")
    base = SC_BASE + wid * RPW

    pltpu.sync_copy(s2t_hbm, c01_ref)

    def row_group(g, carry):
        m0, m1 = carry
        pltpu.sync_copy(adj_hbm.at[pl.ds(base + g * RB, RB), :], abuf_ref)

        accs = [(jnp.zeros((16,), jnp.float32),
                 jnp.zeros((16,), jnp.float32)) for _ in range(RB)]

        def blk_loop(blk, carry_accs):
            out = list(carry_accs)
            for kk in range(BLK // 16):
                off = kk * 16
                c0 = c01_ref[blk, 0, pl.ds(off, 16)]
                c1 = c01_ref[blk, 1, pl.ds(off, 16)]
                for r in range(RB):
                    a = abuf_ref[r, pl.ds(blk * BLK + off, 16)]
                    p0, p1 = out[r]
                    out[r] = (p0 + a * c0, p1 + a * c1)
            return tuple(out)

        accs = lax.fori_loop(0, NBLK, blk_loop, tuple(accs))
        for r in range(RB):
            m0 = jnp.maximum(m0, _lane_sum(accs[r][0]))
            m1 = jnp.maximum(m1, _lane_sum(accs[r][1]))
        return m0, m1

    neg = jnp.full((16,), NEG, jnp.float32)
    m0, m1 = lax.fori_loop(0, RPW // RB, row_group, (neg, neg))

    lane = lax.iota(jnp.int32, 16)
    v = jnp.where(lane == 0, m0, jnp.where(lane == 1, m1, NEG))
    out_ref[...] = v
    pltpu.sync_copy(out_ref, out_hbm.at[wid])


def _combine_body(tcmax_ref, scmax_ref, b2_ref, w3_ref, b3_ref, out_ref):
    scm = jnp.max(scmax_ref[...], axis=0, keepdims=True)
    m = jnp.maximum(tcmax_ref[0:1, :], scm)
    val = jnp.sum((m + b2_ref[...]) * w3_ref[...]) + b3_ref[0, 0]
    out_ref[...] = jnp.full((8, 128), val, jnp.float32)


def kernel(x, adj, W1, b1, W2, b2, W3, b3):
    w2p = jnp.zeros((HPAD, HPAD), jnp.float32).at[:, :2].set(W2)
    b2p = jnp.zeros((1, HPAD), jnp.float32).at[0, :2].set(b2)
    w3p = jnp.zeros((1, HPAD), jnp.float32).at[0, :2].set(W3[:, 0])
    b1r = b1.reshape(1, HPAD)
    b3r = b3.reshape(1, 1)

    s2, s2t = pl.pallas_call(
        _pass1_body,
        grid=(NBLK,),
        in_specs=[
            pl.BlockSpec((N, F_IN), lambda i: (0, 0)),
            pl.BlockSpec((BLK, N), lambda i: (i, 0)),
            pl.BlockSpec((F_IN, HPAD), lambda i: (0, 0)),
            pl.BlockSpec((1, HPAD), lambda i: (0, 0)),
            pl.BlockSpec((HPAD, HPAD), lambda i: (0, 0)),
        ],
        out_specs=[
            pl.BlockSpec((BLK, HPAD), lambda i: (i, 0)),
            pl.BlockSpec((1, 2, BLK), lambda i: (i, 0, 0)),
        ],
        out_shape=[
            jax.ShapeDtypeStruct((N, HPAD), jnp.float32),
            jax.ShapeDtypeStruct((NBLK, 2, BLK), jnp.float32),
        ],
        scratch_shapes=[pltpu.VMEM((N, HPAD), jnp.float32)],
    )(x, adj, W1, b1r, w2p)

    tcmax = pl.pallas_call(
        _pass2_tc_body,
        grid=(TC_NBLK,),
        in_specs=[
            pl.BlockSpec((BLK, N), lambda i: (i, 0)),
            pl.BlockSpec((N, HPAD), lambda i: (0, 0)),
        ],
        out_specs=pl.BlockSpec((8, HPAD), lambda i: (0, 0)),
        out_shape=jax.ShapeDtypeStruct((8, HPAD), jnp.float32),
        scratch_shapes=[pltpu.VMEM((8, HPAD), jnp.float32)],
    )(adj, s2)

    sc_kernel = functools.partial(
        pl.kernel,
        mesh=plsc.VectorSubcoreMesh(core_axis_name="c", subcore_axis_name="s"),
        out_type=jax.ShapeDtypeStruct((NWORK, 16), jnp.float32),
        scratch_types=[
            pltpu.VMEM((NBLK, 2, BLK), jnp.float32),
            pltpu.VMEM((RB, N), jnp.float32),
            pltpu.VMEM((16,), jnp.float32),
        ],
    )(_pass2_sc_body)
    scmax = sc_kernel(adj, s2t)

    out = pl.pallas_call(
        _combine_body,
        in_specs=[
            pl.BlockSpec((8, HPAD), lambda: (0, 0)),
            pl.BlockSpec((NWORK, 16), lambda: (0, 0)),
            pl.BlockSpec((1, HPAD), lambda: (0, 0)),
            pl.BlockSpec((1, HPAD), lambda: (0, 0)),
            pl.BlockSpec((1, 1), lambda: (0, 0)),
        ],
        out_specs=pl.BlockSpec((8, 128), lambda: (0, 0)),
        out_shape=jax.ShapeDtypeStruct((8, 128), jnp.float32),
    )(tcmax, scmax, b2p, w3p, b3r)
    return out[0, 0].reshape(1, 1, 1)


# speedup vs baseline: 1.1528x; 1.1528x over previous
import jax
import jax.numpy as jnp
from jax.experimental import pallas as pl
from jax.experimental.pallas import tpu as pltpu

N = 10000
F_IN = 128
H = 16
BLK = 400
NBLK = N // BLK


def _gcn_body(x_ref, adj_ref, w1_ref, b1_ref, w2_ref, b2_ref, w3_ref, b3_ref,
              out_ref, s1_ref, s2_ref, max_ref):
    p = pl.program_id(0)
    i = pl.program_id(1)

    @pl.when((p == 0) & (i == 0))
    def _():
        s1_ref[...] = jnp.dot(x_ref[...], w1_ref[...],
                              preferred_element_type=jnp.float32)

    @pl.when(p == 0)
    def _():
        t = jnp.dot(adj_ref[...], s1_ref[...],
                    preferred_element_type=jnp.float32)
        h = jnp.maximum(t + b1_ref[...], 0.0)
        s2_ref[pl.ds(i * BLK, BLK), :] = jnp.dot(
            h, w2_ref[...], preferred_element_type=jnp.float32)

    @pl.when(p == 1)
    def _():
        u = jnp.dot(adj_ref[...], s2_ref[...],
                    preferred_element_type=jnp.float32)
        m = jnp.max(u, axis=0, keepdims=True)
        prev = jnp.where(i == 0, jnp.full((1, 2), -3.0e38, jnp.float32),
                         max_ref[...])
        max_ref[...] = jnp.maximum(prev, m)

    @pl.when((p == 1) & (i == NBLK - 1))
    def _():
        pooled = max_ref[...] + b2_ref[...]
        val = (pooled[0, 0] * w3_ref[0, 0] + pooled[0, 1] * w3_ref[1, 0]
               + b3_ref[0, 0])
        out_ref[...] = jnp.full((1, 1, 1), val, jnp.float32)


def kernel(x, adj, W1, b1, W2, b2, W3, b3):
    return pl.pallas_call(
        _gcn_body,
        grid=(2, NBLK),
        in_specs=[
            pl.BlockSpec((N, F_IN), lambda p, i: (0, 0)),
            pl.BlockSpec((BLK, N), lambda p, i: (i, 0)),
            pl.BlockSpec((F_IN, H), lambda p, i: (0, 0)),
            pl.BlockSpec((1, H), lambda p, i: (0, 0)),
            pl.BlockSpec((H, 2), lambda p, i: (0, 0)),
            pl.BlockSpec((1, 2), lambda p, i: (0, 0)),
            pl.BlockSpec((2, 1), lambda p, i: (0, 0)),
            pl.BlockSpec((1, 1), lambda p, i: (0, 0)),
        ],
        out_specs=pl.BlockSpec((1, 1, 1), lambda p, i: (0, 0, 0)),
        out_shape=jax.ShapeDtypeStruct((1, 1, 1), jnp.float32),
        scratch_shapes=[
            pltpu.VMEM((N, H), jnp.float32),
            pltpu.VMEM((N, 2), jnp.float32),
            pltpu.VMEM((1, 2), jnp.float32),
        ],
    )(x, adj, W1, b1.reshape(1, H), W2, b2.reshape(1, 2), W3,
      b3.reshape(1, 1))


# speedup vs baseline: 1.1866x; 1.0292x over previous
import jax
import jax.numpy as jnp
from jax import lax
from jax.experimental import pallas as pl
from jax.experimental.pallas import tpu as pltpu

N = 10000
F_IN = 128
H = 16
BLK = 200
NBLK = N // BLK
CB = 9
CROWS = CB * BLK
S2STEPS = NBLK - CB
CCH = 136
GRID = 2 * NBLK - CB
VLIM = 64 * 1024 * 1024

_NT = (((1,), (1,)), ((), ()))


def _s1t_body(x_ref, w1_ref, s1t_ref):
    s1t_ref[...] = lax.dot_general(
        w1_ref[...], x_ref[...], (((0,), (1,)), ((), ())),
        preferred_element_type=jnp.float32)


def _gcn_body(s1t_ref, adj_ref, b1_ref, w2_ref, b2_ref, w3_ref, b3_ref,
              out_ref, s2_ref, cache_ref, max_ref):
    g = pl.program_id(0)

    @pl.when(g < NBLK)
    def _():
        a = adj_ref[...]
        t = lax.dot_general(a, s1t_ref[...], _NT,
                            preferred_element_type=jnp.float32)
        h = jnp.maximum(t + b1_ref[...], 0.0)
        s2 = jnp.dot(h, w2_ref[...], preferred_element_type=jnp.float32)
        s2_ref[pl.ds(g * BLK, BLK), :] = s2

        @pl.when(g < CB)
        def _():
            cache_ref[pl.ds(g, 1), :, :] = a.astype(jnp.bfloat16).reshape(
                1, BLK, N)

    @pl.when(g >= NBLK)
    def _():
        u = jnp.dot(adj_ref[...], s2_ref[...],
                    preferred_element_type=jnp.float32)
        m = jnp.max(u, axis=0, keepdims=True)

        prev = jnp.where(g == NBLK, jnp.full((1, 2), -3.0e38, jnp.float32),
                         max_ref[...])
        max_ref[...] = jnp.maximum(prev, m)

        c = g - NBLK

        @pl.when(c < CB)
        def _():
            s2b = s2_ref[...].astype(jnp.bfloat16)
            chunk = cache_ref[pl.ds(c, 1), :, :].reshape(BLK, N)
            uc = jnp.dot(chunk, s2b, preferred_element_type=jnp.float32)
            mc = jnp.max(uc, axis=0, keepdims=True)
            max_ref[...] = jnp.maximum(max_ref[...], mc)

    @pl.when(g == GRID - 1)
    def _():
        pooled = max_ref[...] + b2_ref[...]
        val = (pooled[0, 0] * w3_ref[0, 0] + pooled[0, 1] * w3_ref[1, 0]
               + b3_ref[0, 0])
        out_ref[...] = jnp.full((1, 1, 1), val, jnp.float32)


def kernel(x, adj, W1, b1, W2, b2, W3, b3):
    s1t = pl.pallas_call(
        _s1t_body,
        in_specs=[
            pl.BlockSpec((N, F_IN), lambda: (0, 0)),
            pl.BlockSpec((F_IN, H), lambda: (0, 0)),
        ],
        out_specs=pl.BlockSpec((H, N), lambda: (0, 0)),
        out_shape=jax.ShapeDtypeStruct((H, N), jnp.float32),
    )(x, W1)

    def adj_map(g):
        return (jnp.where(g < NBLK, g, g - (NBLK - CB)), 0)

    return pl.pallas_call(
        _gcn_body,
        grid=(GRID,),
        in_specs=[
            pl.BlockSpec((H, N), lambda g: (0, 0)),
            pl.BlockSpec((BLK, N), adj_map),
            pl.BlockSpec((1, H), lambda g: (0, 0)),
            pl.BlockSpec((H, 2), lambda g: (0, 0)),
            pl.BlockSpec((1, 2), lambda g: (0, 0)),
            pl.BlockSpec((2, 1), lambda g: (0, 0)),
            pl.BlockSpec((1, 1), lambda g: (0, 0)),
        ],
        out_specs=pl.BlockSpec((1, 1, 1), lambda g: (0, 0, 0)),
        out_shape=jax.ShapeDtypeStruct((1, 1, 1), jnp.float32),
        scratch_shapes=[
            pltpu.VMEM((N, 2), jnp.float32),
            pltpu.VMEM((CB, BLK, N), jnp.bfloat16),
            pltpu.VMEM((1, 2), jnp.float32),
        ],
        compiler_params=pltpu.CompilerParams(vmem_limit_bytes=VLIM),
    )(s1t, adj, b1.reshape(1, H), W2, b2.reshape(1, 2), W3,
      b3.reshape(1, 1))
